# baseline (speedup 1.0000x reference)
import functools

import jax
import jax.numpy as jnp
from jax import lax
from jax.experimental import pallas as pl
from jax.experimental.pallas import tpu as pltpu
from jax.experimental.pallas import tpu_sc as plsc

EMBED_DIM = 768
NC = 2
NS = 16
NW = NC * NS
CHUNK = 128


@functools.lru_cache(maxsize=None)
def _build(B, D, table_dtype):
    b_per_w = B // NW
    nchunk = b_per_w // CHUNK
    mesh = plsc.VectorSubcoreMesh(core_axis_name="c", subcore_axis_name="s")

    @functools.partial(
        pl.kernel,
        mesh=mesh,
        out_type=jax.ShapeDtypeStruct((B, D), table_dtype),
        scratch_types=[
            pltpu.VMEM((nchunk, CHUNK), jnp.int32),
            pltpu.VMEM((CHUNK, D), table_dtype),
            pltpu.VMEM((CHUNK, D), table_dtype),
            pltpu.SemaphoreType.DMA,
            pltpu.SemaphoreType.DMA,
            pltpu.SemaphoreType.DMA,
            pltpu.SemaphoreType.DMA,
        ],
    )
    def gather_kernel(idx_hbm, table_hbm, out_hbm,
                      idx_v, buf0, buf1, g0, g1, w0, w1):
        wid = lax.axis_index("s") * NC + lax.axis_index("c")
        base = wid * b_per_w
        pltpu.sync_copy(idx_hbm.at[wid], idx_v)

        bufs = (buf0, buf1)
        gsems = (g0, g1)
        wsems = (w0, w1)
        hg = [None, None]
        hw = [None, None]

        hg[0] = pltpu.async_copy(table_hbm.at[idx_v.at[0]], buf0, g0)
        for c in range(nchunk):
            b = c % 2
            if c + 1 < nchunk:
                nb = 1 - b
                if hw[nb] is not None:
                    hw[nb].wait()
                    hw[nb] = None
                hg[nb] = pltpu.async_copy(
                    table_hbm.at[idx_v.at[c + 1]], bufs[nb], gsems[nb])
            hg[b].wait()
            hw[b] = pltpu.async_copy(
                bufs[b], out_hbm.at[pl.ds(base + c * CHUNK, CHUNK)], wsems[b])
        for b in range(2):
            if hw[b] is not None:
                hw[b].wait()

    return gather_kernel


def kernel(input_ids, table):
    batch, seqlen = input_ids.shape
    vocab, dim = table.shape
    B = batch * seqlen
    idx = input_ids.reshape(NW, B // (NW * CHUNK), CHUNK).astype(jnp.int32)
    table_w = lax.bitcast_convert_type(
        table.reshape(vocab, dim // 2, 2), jnp.int32)
    out_w = _build(B, dim // 2, jnp.int32)(idx, table_w)
    out = lax.bitcast_convert_type(out_w, table.dtype)
    return out.reshape(batch, seqlen, dim)
